# Initial kernel scaffold; baseline (speedup 1.0000x reference)
import functools

import jax
import jax.numpy as jnp
from jax import lax
from jax.experimental import pallas as pl
from jax.experimental.pallas import tpu as pltpu
from jax.experimental.pallas import tpu_sc as plsc

_EPS = 1e-12
_L = 16
_C = 128


def _rsqrt16(v):
    i = plsc.bitcast(v, jnp.int32)
    i = jnp.int32(0x5F3759DF) - (i >> 1)
    y = plsc.bitcast(i, jnp.float32)
    for _ in range(3):
        y = y * (1.5 - 0.5 * v * y * y)
    return y


def _make_sc_kernel(N, V, H, P, T):
    info = plsc.get_sparse_core_info()
    NC, NS = info.num_cores, info.num_subcores
    NW = NC * NS
    assert N % (NW * _C) == 0 and H == 128 and P % _C == 0
    per_w = N // NW
    chunks = per_w // _C
    sper = P // _C

    mesh = plsc.VectorSubcoreMesh(core_axis_name="c", subcore_axis_name="s")

    @functools.partial(
        pl.kernel,
        mesh=mesh,
        out_type=jax.ShapeDtypeStruct((N, H), jnp.float32),
        scratch_types=[
            pltpu.VMEM((P, H), jnp.float32),
            pltpu.VMEM((T, H), jnp.float32),
            pltpu.VMEM((H,), jnp.float32),
            pltpu.VMEM((H,), jnp.float32),
            pltpu.VMEM((H,), jnp.float32),
            pltpu.VMEM((_C,), jnp.int32),
            pltpu.VMEM((_C,), jnp.int32),
            pltpu.VMEM((_C, H), jnp.float32),
            pltpu.SemaphoreType.DMA,
        ],
    )
    def k(ids_hbm, tt_hbm, word_hbm, pos_hbm, type_hbm, gamma_hbm, beta_hbm,
          out_hbm, pos_v, ty_v, d_v, g_v, b_v, idx_v, tt_v, rows_v, sem):
        wid = lax.axis_index("s") * NC + lax.axis_index("c")
        base = wid * per_w

        pltpu.sync_copy(pos_hbm, pos_v)
        pltpu.sync_copy(type_hbm, ty_v)
        pltpu.sync_copy(gamma_hbm, g_v)
        pltpu.sync_copy(beta_hbm, b_v)

        for j in range(H // _L):
            sl = pl.ds(j * _L, _L)
            d_v[sl] = ty_v[1, sl] - ty_v[0, sl]

        def bias_pos(s, carry):
            for j in range(H // _L):
                sl = pl.ds(j * _L, _L)
                pos_v[s, sl] = pos_v[s, sl] + ty_v[0, sl]
            return carry

        lax.fori_loop(0, P, bias_pos, 0)

        def row_body(s0):
            def body(i, carry):
                s = s0 + i
                tf = jnp.full((_L,), tt_v[i].astype(jnp.float32))
                xs = []
                acc1 = jnp.zeros((_L,), jnp.float32)
                acc2 = jnp.zeros((_L,), jnp.float32)
                for j in range(H // _L):
                    sl = pl.ds(j * _L, _L)
                    x = rows_v[i, sl] + pos_v[s, sl] + tf * d_v[sl]
                    xs.append(x)
                    acc1 = acc1 + x
                    acc2 = acc2 + x * x
                s1 = jnp.sum(acc1)
                s2 = jnp.sum(acc2)
                mean = s1 * (1.0 / H)
                var = s2 * (1.0 / H) - mean * mean
                r = _rsqrt16(jnp.full((_L,), var + _EPS))
                mv = jnp.full((_L,), mean)
                for j in range(H // _L):
                    sl = pl.ds(j * _L, _L)
                    y = (xs[j] - mv) * r
                    rows_v[i, sl] = y * g_v[sl] + b_v[sl]
                return carry
            return body

        def chunk(g, carry):
            off = base + g * _C
            s0 = lax.rem(g, sper) * _C
            pltpu.sync_copy(ids_hbm.at[pl.ds(off, _C)], idx_v)
            pltpu.sync_copy(tt_hbm.at[pl.ds(off, _C)], tt_v)
            pltpu.async_copy(word_hbm.at[idx_v], rows_v, sem).wait()
            lax.fori_loop(0, _C, row_body(s0), 0)
            pltpu.sync_copy(rows_v, out_hbm.at[pl.ds(off, _C)])
            return carry

        lax.fori_loop(0, chunks, chunk, 0)

    return k


def kernel(input_ids, token_type_ids, word_emb, pos_emb, type_emb, gamma, beta):
    B, S = input_ids.shape
    V, H = word_emb.shape
    P = pos_emb.shape[0]
    T = type_emb.shape[0]
    N = B * S
    ids = input_ids.reshape(N).astype(jnp.int32)
    tts = token_type_ids.reshape(N).astype(jnp.int32)
    k = _make_sc_kernel(N, V, H, P, T)
    out = k(ids, tts,
            word_emb.astype(jnp.float32), pos_emb.astype(jnp.float32),
            type_emb.astype(jnp.float32), gamma.astype(jnp.float32),
            beta.astype(jnp.float32))
    return out.reshape(B, S, H)


# speedup vs baseline: 2.0833x; 2.0833x over previous
import functools

import jax
import jax.numpy as jnp
from jax import lax
from jax.experimental import pallas as pl
from jax.experimental.pallas import tpu as pltpu
from jax.experimental.pallas import tpu_sc as plsc

_EPS = 1e-12
_L = 16
_C = 128


_GATHER_DNUMS = lax.GatherDimensionNumbers(
    offset_dims=(), collapsed_slice_dims=(0,), start_index_map=(0,))


def _permute(v, idx):
    return lax.gather(v, idx[:, None], _GATHER_DNUMS, slice_sizes=(1,),
                      mode=lax.GatherScatterMode.PROMISE_IN_BOUNDS)


def _lanesum(v):
    idx = lax.iota(jnp.int32, _L)
    for sh in (8, 4, 2, 1):
        v = v + _permute(v, idx ^ sh)
    return v


def _splat0(v):
    return _permute(v, jnp.zeros((_L,), jnp.int32))


def _rsqrt16(v):
    i = lax.bitcast_convert_type(v, jnp.int32)
    i = jnp.int32(0x5F3759DF) - (i >> 1)
    y = lax.bitcast_convert_type(i, jnp.float32)
    for _ in range(3):
        y = y * (1.5 - 0.5 * v * y * y)
    return y


def _make_sc_kernel(N, V, H, P, T):
    info = plsc.get_sparse_core_info()
    NC, NS = info.num_cores, info.num_subcores
    NW = NC * NS
    assert N % (NW * _C) == 0 and H == 128 and P % _C == 0
    per_w = N // NW
    chunks = per_w // _C
    sper = P // _C

    mesh = plsc.VectorSubcoreMesh(core_axis_name="c", subcore_axis_name="s")

    @functools.partial(
        pl.kernel,
        mesh=mesh,
        out_type=jax.ShapeDtypeStruct((N, H), jnp.float32),
        scratch_types=[
            pltpu.VMEM((P, H), jnp.float32),
            pltpu.VMEM((T, H), jnp.float32),
            pltpu.VMEM((H,), jnp.float32),
            pltpu.VMEM((H,), jnp.float32),
            pltpu.VMEM((H,), jnp.float32),
            pltpu.VMEM((_C,), jnp.int32),
            pltpu.VMEM((_C + _L,), jnp.int32),
            pltpu.VMEM((_C, H), jnp.float32),
            pltpu.SemaphoreType.DMA,
        ],
    )
    def k(ids_hbm, tt_hbm, word_hbm, pos_hbm, type_hbm, gamma_hbm, beta_hbm,
          out_hbm, pos_v, ty_v, d_v, g_v, b_v, idx_v, tt_v, rows_v, sem):
        wid = lax.axis_index("s") * NC + lax.axis_index("c")
        base = wid * per_w

        pltpu.sync_copy(pos_hbm, pos_v)
        pltpu.sync_copy(type_hbm, ty_v)
        pltpu.sync_copy(gamma_hbm, g_v)
        pltpu.sync_copy(beta_hbm, b_v)

        for j in range(H // _L):
            sl = pl.ds(j * _L, _L)
            d_v[sl] = ty_v[1, sl] - ty_v[0, sl]

        def bias_pos(s, carry):
            for j in range(H // _L):
                sl = pl.ds(j * _L, _L)
                pos_v[s, sl] = pos_v[s, sl] + ty_v[0, sl]
            return carry

        lax.fori_loop(0, P, bias_pos, 0)

        def row_body(s0):
            def body(i, carry):
                s = s0 + i
                tf = _splat0(tt_v[pl.ds(i, _L)]).astype(jnp.float32)
                xs = []
                acc1 = jnp.zeros((_L,), jnp.float32)
                acc2 = jnp.zeros((_L,), jnp.float32)
                for j in range(H // _L):
                    sl = pl.ds(j * _L, _L)
                    x = rows_v[i, sl] + pos_v[s, sl] + tf * d_v[sl]
                    xs.append(x)
                    acc1 = acc1 + x
                    acc2 = acc2 + x * x
                mv = _lanesum(acc1) * (1.0 / H)
                var = _lanesum(acc2) * (1.0 / H) - mv * mv
                r = _rsqrt16(var + _EPS)
                for j in range(H // _L):
                    sl = pl.ds(j * _L, _L)
                    y = (xs[j] - mv) * r
                    rows_v[i, sl] = y * g_v[sl] + b_v[sl]
                return carry
            return body

        def chunk(g, carry):
            off = base + g * _C
            s0 = lax.rem(g, sper) * _C
            pltpu.sync_copy(ids_hbm.at[pl.ds(off, _C)], idx_v)
            pltpu.sync_copy(tt_hbm.at[pl.ds(off, _C)], tt_v.at[pl.ds(0, _C)])
            pltpu.async_copy(word_hbm.at[idx_v], rows_v, sem).wait()
            lax.fori_loop(0, _C, row_body(s0), 0)
            pltpu.sync_copy(rows_v, out_hbm.at[pl.ds(off, _C)])
            return carry

        lax.fori_loop(0, chunks, chunk, 0)

    return k


def kernel(input_ids, token_type_ids, word_emb, pos_emb, type_emb, gamma, beta):
    B, S = input_ids.shape
    V, H = word_emb.shape
    P = pos_emb.shape[0]
    T = type_emb.shape[0]
    N = B * S
    ids = input_ids.reshape(N).astype(jnp.int32)
    tts = token_type_ids.reshape(N).astype(jnp.int32)
    k = _make_sc_kernel(N, V, H, P, T)
    out = k(ids, tts,
            word_emb.astype(jnp.float32), pos_emb.astype(jnp.float32),
            type_emb.astype(jnp.float32), gamma.astype(jnp.float32),
            beta.astype(jnp.float32))
    return out.reshape(B, S, H)
